# baseline (speedup 1.0000x reference)
import functools

import jax
import jax.numpy as jnp
from jax import lax
from jax.experimental import pallas as pl
from jax.experimental.pallas import tpu as pltpu
from jax.experimental.pallas import tpu_sc as plsc

F = 26
FP = 32
D = 64
SLAB = 8


def _gather_alpha_t(alpha_t, idx):
    B = idx.shape[0]
    NC, NS = 2, 16
    NW = NC * NS
    b_per_w = B // NW
    CHUNK = 16
    W = 128
    NCH = b_per_w // CHUNK
    mesh = plsc.VectorSubcoreMesh(core_axis_name="c", subcore_axis_name="s")

    @functools.partial(
        pl.kernel,
        mesh=mesh,
        compiler_params=pltpu.CompilerParams(needs_layout_passes=False),
        out_type=jax.ShapeDtypeStruct((FP, B), jnp.float32),
        scratch_types=[
            pltpu.VMEM((b_per_w,), jnp.int32),
            pltpu.VMEM((CHUNK * F, W), jnp.float32),
            pltpu.VMEM((CHUNK * F, W), jnp.float32),
            pltpu.VMEM((FP, b_per_w), jnp.float32),
            pltpu.SemaphoreType.DMA,
            pltpu.SemaphoreType.DMA,
        ],
    )
    def gather_k(table_hbm, idx_hbm, out_hbm, idx_v, buf0, buf1, out_v, s0, s1):
        wid = lax.axis_index("s") * NC + lax.axis_index("c")
        base = wid * b_per_w
        pltpu.sync_copy(idx_hbm.at[pl.ds(base, b_per_w)], idx_v)

        def fire(c, buf, sem):
            wins = (idx_v[pl.ds(c * CHUNK, CHUNK)] >> 7) << 7
            for j in range(CHUNK):
                col = pl.multiple_of(wins[j], W)
                pltpu.make_async_copy(
                    table_hbm.at[:, pl.ds(col, W)],
                    buf.at[pl.ds(j * F, F)],
                    sem,
                ).start()

        def drain(buf, sem):
            for j in range(CHUNK):
                pltpu.make_async_copy(
                    table_hbm.at[:, pl.ds(0, W)],
                    buf.at[pl.ds(j * F, F)],
                    sem,
                ).wait()

        def extract(c, buf):
            lane = idx_v[pl.ds(c * CHUNK, CHUNK)] & (W - 1)
            r = lax.iota(jnp.int32, 16) * F
            for f in range(F):
                v = plsc.load_gather(buf, [r + f, lane])
                plsc.store_scatter(
                    out_v,
                    [jnp.full((16,), f, jnp.int32),
                     c * CHUNK + lax.iota(jnp.int32, 16)],
                    v,
                )

        def pair_body(p, _):
            c = p * 2
            fire(c + 1, buf1, s1)
            drain(buf0, s0)
            extract(c, buf0)

            @pl.when(p < NCH // 2 - 1)
            def _():
                fire(c + 2, buf0, s0)

            drain(buf1, s1)
            extract(c + 1, buf1)
            return 0

        fire(0, buf0, s0)
        lax.fori_loop(0, NCH // 2, pair_body, 0)
        pltpu.sync_copy(out_v, out_hbm.at[:, pl.ds(base, b_per_w)])

    return gather_k(alpha_t, idx)


def _merge_t(alpha_t8, stack_t, lane_off, block_l=2048):
    Bk = alpha_t8.shape[1]
    ko = lane_off // block_l

    def body(alpha_ref, stack_ref, out_ref):
        a = alpha_ref[...]
        row = lax.broadcasted_iota(jnp.int32, a.shape, 0)
        e = jnp.where(row < F, jnp.exp(a), 0.0)
        s = jnp.sum(e, axis=0, keepdims=True)
        w = e * (1.0 / s)
        acc = w[0:1, :] * stack_ref[0]
        for f in range(1, F):
            acc = acc + w[f : f + 1, :] * stack_ref[f]
        out_ref[...] = acc

    return pl.pallas_call(
        body,
        grid=(Bk // block_l,),
        in_specs=[
            pl.BlockSpec((FP, block_l), lambda i: (0, i)),
            pl.BlockSpec((F, D, block_l), lambda i: (0, 0, i + ko)),
        ],
        out_specs=pl.BlockSpec((D, block_l), lambda i: (0, i)),
        out_shape=jax.ShapeDtypeStruct((D, Bk), jnp.float32),
    )(alpha_t8, stack_t)


def kernel(item_input, stack_embeds, alpha_attention):
    idx = item_input.reshape(-1)
    B = idx.shape[0]
    alpha_t = alpha_attention.T
    stack_t = jnp.transpose(stack_embeds, (1, 2, 0))
    K = 1
    Bk = B // K
    outs = []
    for k in range(K):
        alpha_k = _gather_alpha_t(alpha_t, lax.slice(idx, (k * Bk,), ((k + 1) * Bk,)))
        outs.append(_merge_t(alpha_k, stack_t, k * Bk))
    out_t = jnp.concatenate(outs, axis=1)
    return out_t.T


# speedup vs baseline: 1.0205x; 1.0205x over previous
import functools

import jax
import jax.numpy as jnp
from jax import lax
from jax.experimental import pallas as pl
from jax.experimental.pallas import tpu as pltpu
from jax.experimental.pallas import tpu_sc as plsc

F = 26
FP = 32
D = 64


def _gather_alpha_t(alpha_t, idx):
    B = idx.shape[0]
    NC, NS = 2, 16
    NW = NC * NS
    b_per_w = B // NW
    CHUNK = 16
    W = 128
    NCH = b_per_w // CHUNK
    mesh = plsc.VectorSubcoreMesh(core_axis_name="c", subcore_axis_name="s")

    @functools.partial(
        pl.kernel,
        mesh=mesh,
        compiler_params=pltpu.CompilerParams(needs_layout_passes=False),
        out_type=jax.ShapeDtypeStruct((FP, B), jnp.float32),
        scratch_types=[
            pltpu.VMEM((b_per_w,), jnp.int32),
            pltpu.VMEM((CHUNK * F, W), jnp.float32),
            pltpu.VMEM((CHUNK * F, W), jnp.float32),
            pltpu.VMEM((FP, b_per_w), jnp.float32),
            pltpu.SemaphoreType.DMA,
            pltpu.SemaphoreType.DMA,
        ],
    )
    def gather_k(table_hbm, idx_hbm, out_hbm, idx_v, buf0, buf1, out_v, s0, s1):
        wid = lax.axis_index("s") * NC + lax.axis_index("c")
        base = wid * b_per_w
        pltpu.sync_copy(idx_hbm.at[pl.ds(base, b_per_w)], idx_v)

        def fire(c, buf, sem):
            wins = (idx_v[pl.ds(c * CHUNK, CHUNK)] >> 7) << 7
            for j in range(CHUNK):
                col = pl.multiple_of(wins[j], W)
                pltpu.make_async_copy(
                    table_hbm.at[:, pl.ds(col, W)],
                    buf.at[pl.ds(j * F, F)],
                    sem,
                ).start()

        def drain(buf, sem):
            for j in range(CHUNK):
                pltpu.make_async_copy(
                    table_hbm.at[:, pl.ds(0, W)],
                    buf.at[pl.ds(j * F, F)],
                    sem,
                ).wait()

        def extract(c, buf):
            lane = idx_v[pl.ds(c * CHUNK, CHUNK)] & (W - 1)
            r = lax.iota(jnp.int32, 16) * F
            for f in range(F):
                v = plsc.load_gather(buf, [r + f, lane])
                plsc.store_scatter(
                    out_v,
                    [jnp.full((16,), f, jnp.int32),
                     c * CHUNK + lax.iota(jnp.int32, 16)],
                    v,
                )

        def pair_body(p, _):
            c = p * 2
            fire(c + 1, buf1, s1)
            drain(buf0, s0)
            extract(c, buf0)

            @pl.when(p < NCH // 2 - 1)
            def _():
                fire(c + 2, buf0, s0)

            drain(buf1, s1)
            extract(c + 1, buf1)
            return 0

        fire(0, buf0, s0)
        lax.fori_loop(0, NCH // 2, pair_body, 0)
        pltpu.sync_copy(out_v, out_hbm.at[:, pl.ds(base, b_per_w)])

    return gather_k(alpha_t, idx)


def _merge_t(alpha_t8, stack_t, lane_off, block_l=1024):
    Bk = alpha_t8.shape[1]
    ko = lane_off // block_l

    def body(alpha_ref, stack_ref, out_ref):
        a = alpha_ref[...]
        row = lax.broadcasted_iota(jnp.int32, a.shape, 0)
        e = jnp.where(row < F, jnp.exp(a), 0.0)
        s = jnp.sum(e, axis=0, keepdims=True)
        w = e * (1.0 / s)
        acc = w[0:1, :] * stack_ref[0]
        for f in range(1, F):
            acc = acc + w[f : f + 1, :] * stack_ref[f]
        out_ref[...] = acc

    return pl.pallas_call(
        body,
        grid=(Bk // block_l,),
        in_specs=[
            pl.BlockSpec((FP, block_l), lambda i: (0, i)),
            pl.BlockSpec((F, D, block_l), lambda i: (0, 0, i + ko)),
        ],
        out_specs=pl.BlockSpec((D, block_l), lambda i: (0, i)),
        out_shape=jax.ShapeDtypeStruct((D, Bk), jnp.float32),
    )(alpha_t8, stack_t)


def kernel(item_input, stack_embeds, alpha_attention):
    idx = item_input.reshape(-1)
    B = idx.shape[0]
    alpha_t = alpha_attention.T
    stack_t = jnp.transpose(stack_embeds, (1, 2, 0))
    K = 1
    Bk = B // K
    outs = []
    for k in range(K):
        alpha_k = _gather_alpha_t(alpha_t, lax.slice(idx, (k * Bk,), ((k + 1) * Bk,)))
        outs.append(_merge_t(alpha_k, stack_t, k * Bk))
    out_t = jnp.concatenate(outs, axis=1)
    return out_t.T
